# baseline (speedup 1.0000x reference)
import functools

import jax
import jax.numpy as jnp
from jax import lax
from jax.experimental import pallas as pl
from jax.experimental.pallas import tpu as pltpu
from jax.experimental.pallas import tpu_sc as plsc

_NC = 2
_NS = 16
_NW = _NC * _NS


def _sc_kernel(n, d, rw, ch):
    nch = rw // ch
    mesh = plsc.VectorSubcoreMesh(core_axis_name="c", subcore_axis_name="s")

    @functools.partial(
        pl.kernel,
        out_type=jax.ShapeDtypeStruct((n * d,), jnp.float32),
        mesh=mesh,
        scratch_types=[
            pltpu.VMEM((16,), jnp.int32),
            pltpu.VMEM((8 * rw,), jnp.float32),
            pltpu.VMEM((ch * d,), jnp.float32),
            pltpu.VMEM((ch * d,), jnp.float32),
            pltpu.SemaphoreType.DMA,
            pltpu.SemaphoreType.DMA,
            pltpu.SemaphoreType.DMA,
            pltpu.SemaphoreType.DMA,
        ],
    )
    def k(xf_hbm, w2f_hbm, tmv_hbm, out_hbm,
          tmbuf, wbuf, xb0, xb1, si0, si1, so0, so1):
        cid = lax.axis_index("c")
        sid = lax.axis_index("s")
        wid = sid * _NC + cid
        base = wid * rw

        pltpu.sync_copy(tmv_hbm, tmbuf)
        pltpu.sync_copy(w2f_hbm.at[pl.ds(8 * base, 8 * rw)], wbuf)
        tm = tmbuf[...]

        xbufs = (xb0, xb1)
        sin = (si0, si1)
        sout = (so0, so1)

        def compute(c, xb):
            def grp_body(g2, _):
                lr = c * ch + 16 * g2
                for j in range(8):
                    v = wbuf[pl.ds(8 * lr + 16 * j, 16)]
                    for h in range(2):
                        splat = v.at[tm + 8 * h].get(mode="promise_in_bounds")
                        off = d * (16 * g2 + 2 * j + h)
                        for q in range(d // 16):
                            sl = pl.ds(off + 16 * q, 16)
                            xb[sl] = xb[sl] - splat
                return 0

            lax.fori_loop(0, ch // 16, grp_body, 0)

        in_copies = [None] * nch
        out_copies = [None] * nch
        for c in range(min(2, nch)):
            in_copies[c] = pltpu.make_async_copy(
                xf_hbm.at[pl.ds(d * (base + c * ch), d * ch)],
                xbufs[c % 2], sin[c % 2],
            )
            in_copies[c].start()
        for c in range(nch):
            b = c % 2
            in_copies[c].wait()
            compute(c, xbufs[b])
            out_copies[c] = pltpu.make_async_copy(
                xbufs[b], out_hbm.at[pl.ds(d * (base + c * ch), d * ch)],
                sout[b],
            )
            out_copies[c].start()
            if c + 2 < nch:
                out_copies[c].wait()
                in_copies[c + 2] = pltpu.make_async_copy(
                    xf_hbm.at[pl.ds(d * (base + (c + 2) * ch), d * ch)],
                    xbufs[b], sin[b],
                )
                in_copies[c + 2].start()
        for c in range(max(0, nch - 2), nch):
            out_copies[c].wait()

    return k


def kernel(x, weight, token_type):
    n, d = x.shape
    rw = n // _NW
    ch = 128
    t = jnp.asarray(token_type, jnp.int32)
    t0 = (t // 8) * 8
    w8 = lax.dynamic_slice(weight, (jnp.int32(0), t0), (n, 8))
    w2f = w8.reshape(n * 8)
    tmv = jnp.full((16,), t % 8, jnp.int32)
    xf = x.reshape(n * d)
    out = _sc_kernel(n, d, rw, ch)(xf, w2f, tmv)
    return out.reshape(n, d)


# speedup vs baseline: 1.6923x; 1.6923x over previous
import jax
import jax.numpy as jnp
from jax import lax
from jax.experimental import pallas as pl
from jax.experimental.pallas import tpu as pltpu


def _body(tt_ref, x_ref, wt_ref, o_ref):
    tm = tt_ref[0]
    wt = wt_ref[...]
    bn = x_ref.shape[0]
    bnk = wt.shape[1]

    qi = jax.lax.broadcasted_iota(jnp.int32, (16, 128), 0)
    li = jax.lax.broadcasted_iota(jnp.int32, (16, 128), 1)
    m5 = (li == 8 * qi + tm).astype(jnp.float32)
    w16 = jax.lax.dot(m5, wt, precision=jax.lax.Precision.HIGHEST)

    drep = pltpu.repeat(w16, bn // 16, axis=0)
    ksel = jax.lax.broadcasted_iota(jnp.int32, (bn, bnk), 1)
    rdiv = jax.lax.broadcasted_iota(jnp.int32, (bn, bnk), 0) // 16
    col = jnp.sum(jnp.where(ksel == rdiv, drep, 0.0), axis=1, keepdims=True)
    o_ref[...] = x_ref[...] - col


def kernel(x, weight, token_type):
    n, d = x.shape
    bn = 2048
    bnk = bn // 16
    t = jnp.asarray(token_type, jnp.int32)
    t0 = (t // 8) * 8
    w8 = lax.dynamic_slice(weight, (jnp.int32(0), t0), (n, 8))
    w2t = w8.reshape(n // 16, 128).T
    tm = (t % 8).reshape(1)
    return pl.pallas_call(
        _body,
        grid=(n // bn,),
        in_specs=[
            pl.BlockSpec(memory_space=pltpu.SMEM),
            pl.BlockSpec((bn, d), lambda i: (i, 0)),
            pl.BlockSpec((128, bnk), lambda i: (0, i)),
        ],
        out_specs=pl.BlockSpec((bn, d), lambda i: (i, 0)),
        out_shape=jax.ShapeDtypeStruct((n, d), jnp.float32),
    )(tm, x, w2t)
